# baseline (speedup 1.0000x reference)
import jax
import jax.numpy as jnp
from jax import lax
from jax.experimental import pallas as pl
from jax.experimental.pallas import tpu as pltpu
from jax.experimental.pallas import tpu_sc as plsc

B, K, D = 1024, 50, 32
N = 1000000
KPI = 56
KPA = 64
NC, NS = 2, 16
NW = NC * NS
BPW = B // NW
HALF = D // 2
SUP = 4 * D
NBUF = 4


def _dis_loss_body(emb_hbm, table_hbm, attr_hbm, idx_hbm, start_hbm, out_hbm,
                   embb_v, attr_v, idx_v, start_v, rows_v, o_v,
                   sem0, sem1, sem2, sem3):
    sems = [sem0, sem1, sem2, sem3]
    wid = lax.axis_index("s") * NC + lax.axis_index("c")
    base = wid * BPW
    pltpu.sync_copy(emb_hbm.at[pl.ds(base, BPW)], embb_v)
    pltpu.sync_copy(attr_hbm.at[pl.ds(base, BPW)], attr_v)
    pltpu.sync_copy(idx_hbm.at[pl.ds(base, BPW)], idx_v)
    pltpu.sync_copy(start_hbm.at[pl.ds(base, BPW)], start_v)

    def fire(b, slot):
        pltpu.async_copy(table_hbm.at[idx_v.at[b]], rows_v.at[slot], sems[slot])

    for j in range(NBUF):
        fire(j, j)

    def outer(g, carry):
        acc_lo, acc_hi = carry
        for j in range(NBUF):
            b = g * NBUF + j
            pltpu.make_async_copy(
                table_hbm.at[idx_v.at[b]], rows_v.at[j], sems[j]).wait()
            x_lo = embb_v[b, 0:HALF]
            x_hi = embb_v[b, HALF:D]
            for g2 in range(KPA // HALF):
                av = attr_v[b, g2 * HALF:(g2 + 1) * HALF]
                sv = start_v[b, g2 * HALF:(g2 + 1) * HALF]
                for kk in range(HALF):
                    k = g2 * HALF + kk
                    if k >= KPI:
                        break
                    a = av[kk]
                    st = sv[kk]
                    d_lo = rows_v[j, k, pl.ds(st, HALF)] - x_lo
                    d_hi = rows_v[j, k, pl.ds(st + HALF, HALF)] - x_hi
                    acc_lo = acc_lo + a * (d_lo * d_lo)
                    acc_hi = acc_hi + a * (d_hi * d_hi)

            @pl.when(b + NBUF < BPW)
            def _():
                fire(b + NBUF, j)

        return (acc_lo, acc_hi)

    z = jnp.zeros((HALF,), jnp.float32)
    acc_lo, acc_hi = lax.fori_loop(0, BPW // NBUF, outer, (z, z))
    o_v[...] = acc_lo + acc_hi
    pltpu.sync_copy(o_v, out_hbm.at[wid])


def kernel(emb_batch, embedding, attr_sim, indices, beta):
    del beta
    table2 = embedding.reshape(N // 4, SUP)
    idx_p = jnp.pad(indices, ((0, 0), (0, KPI - K)))
    sup = idx_p // 4
    start = (idx_p % 4) * D
    start = jnp.pad(start, ((0, 0), (0, KPA - KPI)))
    attr_p = jnp.pad(attr_sim, ((0, 0), (0, KPA - K)))
    mesh = plsc.VectorSubcoreMesh(core_axis_name="c", subcore_axis_name="s")
    out = pl.kernel(
        _dis_loss_body,
        out_type=jax.ShapeDtypeStruct((NW, HALF), jnp.float32),
        mesh=mesh,
        compiler_params=pltpu.CompilerParams(use_tc_tiling_on_sc=False),
        scratch_types=[
            pltpu.VMEM((BPW, D), jnp.float32),
            pltpu.VMEM((BPW, KPA), jnp.float32),
            pltpu.VMEM((BPW, KPI), jnp.int32),
            pltpu.VMEM((BPW, KPA), jnp.int32),
            pltpu.VMEM((NBUF, KPI, SUP), jnp.float32),
            pltpu.VMEM((HALF,), jnp.float32),
            pltpu.SemaphoreType.DMA,
            pltpu.SemaphoreType.DMA,
            pltpu.SemaphoreType.DMA,
            pltpu.SemaphoreType.DMA,
        ],
    )(emb_batch, table2, attr_p, sup, start)
    return jnp.sum(out) / jnp.float32(B)


# speedup vs baseline: 1.0001x; 1.0001x over previous
import jax
import jax.numpy as jnp
from jax import lax
from jax.experimental import pallas as pl
from jax.experimental.pallas import tpu as pltpu
from jax.experimental.pallas import tpu_sc as plsc

B, K, D = 1024, 50, 32
N = 1000000
KPI = 56
KPA = 64
NC, NS = 2, 16
NW = NC * NS
BPW = B // NW
HALF = D // 2
SUP = 4 * D
NBUF = 4


def _dis_loss_body(emb_hbm, table_hbm, attr_hbm, idx_hbm, start_hbm, out_hbm,
                   embb_v, attr_v, idx_v, start_v, rows_v, o_v,
                   sem0, sem1, sem2, sem3):
    sems = [sem0, sem1, sem2, sem3]
    wid = lax.axis_index("s") * NC + lax.axis_index("c")
    pltpu.sync_copy(emb_hbm.at[pl.ds(wid * (BPW * D), BPW * D)], embb_v)
    pltpu.sync_copy(attr_hbm.at[pl.ds(wid * (BPW * KPA), BPW * KPA)], attr_v)
    pltpu.sync_copy(idx_hbm.at[pl.ds(wid * (BPW * KPI), BPW * KPI)], idx_v)
    pltpu.sync_copy(start_hbm.at[pl.ds(wid * (BPW * KPA), BPW * KPA)], start_v)

    def fire(b, slot):
        pltpu.async_copy(table_hbm.at[idx_v.at[pl.ds(b * KPI, KPI)]],
                         rows_v.at[slot], sems[slot])

    for j in range(NBUF):
        fire(j, j)

    def outer(g, carry):
        acc_lo, acc_hi = carry
        for j in range(NBUF):
            b = g * NBUF + j
            pltpu.make_async_copy(
                table_hbm.at[idx_v.at[pl.ds(b * KPI, KPI)]],
                rows_v.at[j], sems[j]).wait()
            x_lo = embb_v[pl.ds(b * D, HALF)]
            x_hi = embb_v[pl.ds(b * D + HALF, HALF)]
            for g2 in range(KPA // HALF):
                av = attr_v[pl.ds(b * KPA + g2 * HALF, HALF)]
                sv = start_v[pl.ds(b * KPA + g2 * HALF, HALF)]
                for kk in range(HALF):
                    k = g2 * HALF + kk
                    if k >= KPI:
                        break
                    a = av[kk]
                    st = sv[kk]
                    d_lo = rows_v[j, k, pl.ds(st, HALF)] - x_lo
                    d_hi = rows_v[j, k, pl.ds(st + HALF, HALF)] - x_hi
                    acc_lo = acc_lo + a * (d_lo * d_lo)
                    acc_hi = acc_hi + a * (d_hi * d_hi)

            @pl.when(b + NBUF < BPW)
            def _():
                fire(b + NBUF, j)

        return (acc_lo, acc_hi)

    z = jnp.zeros((HALF,), jnp.float32)
    acc_lo, acc_hi = lax.fori_loop(0, BPW // NBUF, outer, (z, z))
    o_v[...] = acc_lo + acc_hi
    pltpu.sync_copy(o_v, out_hbm.at[pl.ds(wid * HALF, HALF)])


def kernel(emb_batch, embedding, attr_sim, indices, beta):
    del beta
    table2 = embedding.reshape(N // 4, SUP)
    idx_p = jnp.pad(indices, ((0, 0), (0, KPI - K)))
    sup = (idx_p // 4).reshape(-1)
    start = (idx_p % 4) * D
    start = jnp.pad(start, ((0, 0), (0, KPA - KPI))).reshape(-1)
    attr_p = jnp.pad(attr_sim, ((0, 0), (0, KPA - K))).reshape(-1)
    emb_flat = emb_batch.reshape(-1)
    mesh = plsc.VectorSubcoreMesh(core_axis_name="c", subcore_axis_name="s")
    out = pl.kernel(
        _dis_loss_body,
        out_type=jax.ShapeDtypeStruct((NW * HALF,), jnp.float32),
        mesh=mesh,
        scratch_types=[
            pltpu.VMEM((BPW * D,), jnp.float32),
            pltpu.VMEM((BPW * KPA,), jnp.float32),
            pltpu.VMEM((BPW * KPI,), jnp.int32),
            pltpu.VMEM((BPW * KPA,), jnp.int32),
            pltpu.VMEM((NBUF, KPI, SUP), jnp.float32),
            pltpu.VMEM((HALF,), jnp.float32),
            pltpu.SemaphoreType.DMA,
            pltpu.SemaphoreType.DMA,
            pltpu.SemaphoreType.DMA,
            pltpu.SemaphoreType.DMA,
        ],
    )(emb_flat, table2, attr_p, sup, start)
    return jnp.sum(out) / jnp.float32(B)


# speedup vs baseline: 1.3123x; 1.3121x over previous
import jax
import jax.numpy as jnp
from jax import lax
from jax.experimental import pallas as pl
from jax.experimental.pallas import tpu as pltpu
from jax.experimental.pallas import tpu_sc as plsc

B, K, D = 1024, 50, 32
N = 1000000
KPI = 56
KPA = 64
NC, NS = 2, 16
NW = NC * NS
BPW = B // NW
HALF = D // 2


def _dis_loss_body(emb_hbm, table_hbm, attr_hbm, idx_hbm, out_hbm,
                   embb_v, attr_v, idx_v, rows_v, o_v, sem):
    wid = lax.axis_index("s") * NC + lax.axis_index("c")
    pltpu.sync_copy(emb_hbm.at[pl.ds(wid * (BPW * D), BPW * D)], embb_v)
    pltpu.sync_copy(attr_hbm.at[pl.ds(wid * (BPW * KPA), BPW * KPA)], attr_v)
    pltpu.sync_copy(idx_hbm.at[pl.ds(wid * (BPW * KPI), BPW * KPI)], idx_v)

    for b in range(BPW):
        pltpu.async_copy(table_hbm.at[idx_v.at[pl.ds(b * KPI, KPI)]],
                         rows_v.at[b], sem)
    for b in range(BPW):
        pltpu.make_async_copy(
            table_hbm.at[idx_v.at[pl.ds(b * KPI, KPI)]],
            rows_v.at[b], sem).wait()

    def b_loop(b, carry):
        acc_lo, acc_hi = carry
        x_lo = embb_v[pl.ds(b * D, HALF)]
        x_hi = embb_v[pl.ds(b * D + HALF, HALF)]
        for g2 in range(KPA // HALF):
            av = attr_v[pl.ds(b * KPA + g2 * HALF, HALF)]
            for kk in range(HALF):
                k = g2 * HALF + kk
                if k >= KPI:
                    break
                a = av[kk]
                d_lo = rows_v[b, k, 0:HALF] - x_lo
                d_hi = rows_v[b, k, HALF:D] - x_hi
                acc_lo = acc_lo + a * (d_lo * d_lo)
                acc_hi = acc_hi + a * (d_hi * d_hi)
        return (acc_lo, acc_hi)

    z = jnp.zeros((HALF,), jnp.float32)
    acc_lo, acc_hi = lax.fori_loop(0, BPW, b_loop, (z, z))
    o_v[...] = acc_lo + acc_hi
    pltpu.sync_copy(o_v, out_hbm.at[pl.ds(wid * HALF, HALF)])


def kernel(emb_batch, embedding, attr_sim, indices, beta):
    del beta
    idx_p = jnp.pad(indices, ((0, 0), (0, KPI - K))).reshape(-1)
    attr_p = jnp.pad(attr_sim, ((0, 0), (0, KPA - K))).reshape(-1)
    emb_flat = emb_batch.reshape(-1)
    mesh = plsc.VectorSubcoreMesh(core_axis_name="c", subcore_axis_name="s")
    out = pl.kernel(
        _dis_loss_body,
        out_type=jax.ShapeDtypeStruct((NW * HALF,), jnp.float32),
        mesh=mesh,
        compiler_params=pltpu.CompilerParams(use_tc_tiling_on_sc=False),
        scratch_types=[
            pltpu.VMEM((BPW * D,), jnp.float32),
            pltpu.VMEM((BPW * KPA,), jnp.float32),
            pltpu.VMEM((BPW * KPI,), jnp.int32),
            pltpu.VMEM((BPW, KPI, D), jnp.float32),
            pltpu.VMEM((HALF,), jnp.float32),
            pltpu.SemaphoreType.DMA,
        ],
    )(emb_flat, embedding, attr_p, idx_p)
    return jnp.sum(out) / jnp.float32(B)
